# Initial kernel scaffold; baseline (speedup 1.0000x reference)
import jax
import jax.numpy as jnp
from jax.experimental import pallas as pl


def kernel(input_ids, attention_mask, embedding_weight):
    raise NotImplementedError("write your pallas kernel here")


# speedup vs baseline: 14.5539x; 14.5539x over previous
import jax
import jax.numpy as jnp
from jax.experimental import pallas as pl

B, L, V, D = 1024, 50, 1000, 768
VP = 1024
BB = 128


def _body(ids_ref, mask_ref, w_ref, out_ref):
    ids = ids_ref[...]
    mask = mask_ref[...].astype(jnp.float32)
    iota_v = jax.lax.broadcasted_iota(jnp.int32, (BB, VP), 1)
    hist = jnp.zeros((BB, VP), jnp.float32)
    for l in range(L):
        idv = ids[:, l][:, None]
        mv = mask[:, l][:, None]
        hist = hist + jnp.where(iota_v == idv, mv, 0.0)
    count = jnp.sum(hist, axis=1, keepdims=True)
    acc = jnp.dot(hist, w_ref[...], preferred_element_type=jnp.float32)
    out_ref[...] = acc / jnp.maximum(count, 1e-9)


def kernel(input_ids, attention_mask, embedding_weight):
    ids = input_ids.astype(jnp.int32)
    mask = attention_mask.astype(jnp.int32)
    wp = jnp.pad(embedding_weight, ((0, VP - V), (0, 0)))
    return pl.pallas_call(
        _body,
        grid=(B // BB,),
        in_specs=[
            pl.BlockSpec((BB, L), lambda i: (i, 0)),
            pl.BlockSpec((BB, L), lambda i: (i, 0)),
            pl.BlockSpec((VP, D), lambda i: (0, 0)),
        ],
        out_specs=pl.BlockSpec((BB, D), lambda i: (i, 0)),
        out_shape=jax.ShapeDtypeStruct((B, D), jnp.float32),
    )(ids, mask, wp)
